# baseline (speedup 1.0000x reference)
import functools

import jax
import jax.numpy as jnp
from jax.experimental import pallas as pl
from jax.experimental.pallas import tpu as pltpu

_SIGMA = 1.0
_BLK = 128


def _proxy_nca_body(xs_ref, ys_ref, prox_ref, out_ref, p2_ref, pn2_ref):
    i = pl.program_id(0)
    blk = xs_ref.shape[0]
    C = prox_ref.shape[0]

    @pl.when(i == 0)
    def _init():
        prox = prox_ref[:]
        n2 = jnp.sum(prox * prox, axis=1, keepdims=True)
        scale = 2.0 / jnp.maximum(jnp.sqrt(n2), 1e-12)
        p2 = prox * scale
        p2_ref[:] = p2
        pn2_ref[:] = jax.lax.dot_general(
            jnp.ones((1, p2.shape[1]), jnp.float32), p2 * p2,
            (((1,), (1,)), ((), ())),
            preferred_element_type=jnp.float32) * 0.25
        out_ref[:] = jnp.zeros((1, 1), jnp.float32)

    g2 = jax.lax.dot_general(
        xs_ref[:], p2_ref[:], (((1,), (1,)), ((), ())),
        preferred_element_type=jnp.float32)
    s = (g2 - pn2_ref[:]) * (1.0 / _SIGMA)

    m = jnp.max(s, axis=1, keepdims=True)
    lse = m + jnp.log(jnp.sum(jnp.exp(s - m), axis=1, keepdims=True))

    col = jax.lax.broadcasted_iota(jnp.int32, (blk, C), 1)
    s_pos = jnp.sum(jnp.where(col == ys_ref[:], s, 0.0),
                    axis=1, keepdims=True)

    out_ref[:] += jnp.sum(lse - s_pos, axis=(0, 1), keepdims=True) * (
        1.0 / (blk * pl.num_programs(0)))


@functools.partial(jax.jit, static_argnames=())
def kernel(xs, ys, proxies):
    B, D = xs.shape
    C = proxies.shape[0]
    n_blk = B // _BLK
    out = pl.pallas_call(
        _proxy_nca_body,
        grid=(n_blk,),
        in_specs=[
            pl.BlockSpec((_BLK, D), lambda i: (i, 0)),
            pl.BlockSpec((_BLK, 1), lambda i: (i, 0)),
            pl.BlockSpec((C, D), lambda i: (0, 0)),
        ],
        out_specs=pl.BlockSpec((1, 1), lambda i: (0, 0)),
        out_shape=jax.ShapeDtypeStruct((1, 1), jnp.float32),
        scratch_shapes=[
            pltpu.VMEM((C, D), jnp.float32),
            pltpu.VMEM((1, C), jnp.float32),
        ],
    )(xs, ys.reshape(B, 1), proxies)
    return out[0, 0]


# speedup vs baseline: 1.2597x; 1.2597x over previous
import functools

import jax
import jax.numpy as jnp
from jax.experimental import pallas as pl
from jax.experimental.pallas import tpu as pltpu

_SIGMA = 1.0
_BLK = 256


def _proxy_nca_body(xs_ref, ys_ref, prox_ref, out_ref, p2_ref, pn2_ref):
    i = pl.program_id(0)
    blk = xs_ref.shape[0]
    C = prox_ref.shape[0]

    @pl.when(i == 0)
    def _init():
        prox = prox_ref[:]
        n2 = jnp.sum(prox * prox, axis=1, keepdims=True)
        scale = 2.0 / jnp.maximum(jnp.sqrt(n2), 1e-12)
        p2 = prox * scale
        p2_ref[:] = p2
        pn2_ref[:] = jax.lax.dot_general(
            jnp.ones((1, p2.shape[1]), jnp.float32), p2 * p2,
            (((1,), (1,)), ((), ())),
            preferred_element_type=jnp.float32) * 0.25
        out_ref[:] = jnp.zeros((1, 1), jnp.float32)

    g2 = jax.lax.dot_general(
        xs_ref[:], p2_ref[:], (((1,), (1,)), ((), ())),
        preferred_element_type=jnp.float32)
    s = (g2 - pn2_ref[:]) * (1.0 / _SIGMA)

    m = jnp.max(s, axis=1, keepdims=True)
    lse = m + jnp.log(jnp.sum(jnp.exp(s - m), axis=1, keepdims=True))

    col = jax.lax.broadcasted_iota(jnp.int32, (blk, C), 1)
    s_pos = jnp.sum(jnp.where(col == ys_ref[:], s, 0.0),
                    axis=1, keepdims=True)

    out_ref[:] += jnp.sum(lse - s_pos, axis=(0, 1), keepdims=True) * (
        1.0 / (blk * pl.num_programs(0)))


@functools.partial(jax.jit, static_argnames=())
def kernel(xs, ys, proxies):
    B, D = xs.shape
    C = proxies.shape[0]
    n_blk = B // _BLK
    out = pl.pallas_call(
        _proxy_nca_body,
        grid=(n_blk,),
        in_specs=[
            pl.BlockSpec((_BLK, D), lambda i: (i, 0)),
            pl.BlockSpec((_BLK, 1), lambda i: (i, 0)),
            pl.BlockSpec((C, D), lambda i: (0, 0)),
        ],
        out_specs=pl.BlockSpec((1, 1), lambda i: (0, 0)),
        out_shape=jax.ShapeDtypeStruct((1, 1), jnp.float32),
        scratch_shapes=[
            pltpu.VMEM((C, D), jnp.float32),
            pltpu.VMEM((1, C), jnp.float32),
        ],
    )(xs, ys.reshape(B, 1), proxies)
    return out[0, 0]


# speedup vs baseline: 1.3570x; 1.0773x over previous
import functools

import jax
import jax.numpy as jnp
from jax.experimental import pallas as pl

_SIGMA = 1.0


def _proxy_nca_body(xs_ref, ys_ref, prox_ref, out_ref):
    B = xs_ref.shape[0]
    C = prox_ref.shape[0]

    prox = prox_ref[:]
    n2 = jnp.sum(prox * prox, axis=1, keepdims=True)
    scale = 2.0 / jnp.maximum(jnp.sqrt(n2), 1e-12)
    p2 = prox * scale

    g2 = jax.lax.dot_general(
        xs_ref[:], p2, (((1,), (1,)), ((), ())),
        preferred_element_type=jnp.float32)
    pn2_row = jax.lax.dot_general(
        jnp.ones((1, p2.shape[1]), jnp.float32), p2 * p2,
        (((1,), (1,)), ((), ())),
        preferred_element_type=jnp.float32) * 0.25
    s = (g2 - pn2_row) * (1.0 / _SIGMA)

    m = jnp.max(s, axis=1, keepdims=True)
    lse = m + jnp.log(jnp.sum(jnp.exp(s - m), axis=1, keepdims=True))

    col = jax.lax.broadcasted_iota(jnp.int32, (B, C), 1)
    s_pos = jnp.sum(jnp.where(col == ys_ref[:], s, 0.0),
                    axis=1, keepdims=True)

    out_ref[:, :] = jnp.sum(lse - s_pos, axis=(0, 1), keepdims=True) * (1.0 / B)


@functools.partial(jax.jit, static_argnames=())
def kernel(xs, ys, proxies):
    out = pl.pallas_call(
        _proxy_nca_body,
        out_shape=jax.ShapeDtypeStruct((1, 1), jnp.float32),
    )(xs, ys.reshape(xs.shape[0], 1), proxies)
    return out[0, 0]


# speedup vs baseline: 1.3859x; 1.0213x over previous
import functools

import jax
import jax.numpy as jnp
from jax.experimental import pallas as pl

_SIGMA = 1.0


def _proxy_nca_body(xs_ref, ys_ref, prox_ref, out_ref):
    B = xs_ref.shape[0]
    C = prox_ref.shape[0]

    prox = prox_ref[:]
    raw = jax.lax.dot_general(
        xs_ref[:], prox, (((1,), (1,)), ((), ())),
        preferred_element_type=jnp.float32)
    n2_row = jax.lax.dot_general(
        jnp.ones((1, prox.shape[1]), jnp.float32), prox * prox,
        (((1,), (1,)), ((), ())),
        preferred_element_type=jnp.float32)
    inv = 1.0 / jnp.maximum(jnp.sqrt(n2_row), 1e-12)
    pn2_row = n2_row * (inv * inv)
    s = (raw * (2.0 * inv) - pn2_row) * (1.0 / _SIGMA)

    m = jnp.max(s, axis=1, keepdims=True)
    lse = m + jnp.log(jnp.sum(jnp.exp(s - m), axis=1, keepdims=True))

    col = jax.lax.broadcasted_iota(jnp.int32, (B, C), 1)
    s_pos = jnp.sum(jnp.where(col == ys_ref[:], s, 0.0),
                    axis=1, keepdims=True)

    out_ref[:, :] = jnp.sum(lse - s_pos, axis=(0, 1), keepdims=True) * (1.0 / B)


@functools.partial(jax.jit, static_argnames=())
def kernel(xs, ys, proxies):
    out = pl.pallas_call(
        _proxy_nca_body,
        out_shape=jax.ShapeDtypeStruct((1, 1), jnp.float32),
    )(xs, ys.reshape(xs.shape[0], 1), proxies)
    return out[0, 0]


# speedup vs baseline: 2.5050x; 1.8075x over previous
import functools

import jax
import jax.numpy as jnp
from jax.experimental import pallas as pl

_SIGMA = 1.0


def _proxy_nca_body(xs_ref, ys_ref, prox_ref, out_ref):
    B = xs_ref.shape[0]
    C = prox_ref.shape[0]

    prox = prox_ref[:]
    raw = jax.lax.dot_general(
        prox, xs_ref[:], (((1,), (1,)), ((), ())),
        preferred_element_type=jnp.float32)
    n2 = jnp.sum(prox * prox, axis=1, keepdims=True)
    inv = 1.0 / jnp.maximum(jnp.sqrt(n2), 1e-12)
    pn2 = n2 * (inv * inv)
    s = (raw * (2.0 * inv) - pn2) * (1.0 / _SIGMA)

    m = jnp.max(s, axis=0, keepdims=True)
    lse = m + jnp.log(jnp.sum(jnp.exp(s - m), axis=0, keepdims=True))

    row = jax.lax.broadcasted_iota(jnp.int32, (C, B), 0)
    s_pos = jnp.sum(jnp.where(row == ys_ref[:], s, 0.0),
                    axis=0, keepdims=True)

    out_ref[:, :] = jnp.sum(lse - s_pos, axis=(0, 1), keepdims=True) * (1.0 / B)


@functools.partial(jax.jit, static_argnames=())
def kernel(xs, ys, proxies):
    out = pl.pallas_call(
        _proxy_nca_body,
        out_shape=jax.ShapeDtypeStruct((1, 1), jnp.float32),
    )(xs, ys.reshape(1, xs.shape[0]), proxies)
    return out[0, 0]
